# baseline (speedup 1.0000x reference)
import jax
import jax.numpy as jnp
from jax import lax
from jax.experimental import pallas as pl
from jax.experimental.pallas import tpu as pltpu

_EPS = 1e-7
_BLK = 1200


def _loss_kernel(td_hbm, ts_hbm, od_hbm, os_hbm, out_ref,
                 td_b, ts_b, od_b, os_b, sems, acc_ref):
    n = td_hbm.shape[1]
    spb = n // _BLK
    total = td_hbm.shape[0] * spb

    def copies(step):
        slot = lax.rem(step, 2)
        b = step // spb
        r0 = lax.rem(step, spb) * _BLK
        return [
            pltpu.make_async_copy(td_hbm.at[b, pl.ds(r0, _BLK), :],
                                  td_b.at[slot], sems.at[slot, 0]),
            pltpu.make_async_copy(od_hbm.at[b, pl.ds(r0, _BLK), :],
                                  od_b.at[slot], sems.at[slot, 1]),
            pltpu.make_async_copy(ts_hbm.at[b, pl.ds(r0, _BLK), :],
                                  ts_b.at[slot], sems.at[slot, 2]),
            pltpu.make_async_copy(os_hbm.at[b, pl.ds(r0, _BLK), :],
                                  os_b.at[slot], sems.at[slot, 3]),
        ]

    acc_ref[0] = 0.0
    acc_ref[1] = 0.0
    acc_ref[2] = 0.0
    for cp in copies(0):
        cp.start()

    def body(s, carry):
        @pl.when(s + 1 < total)
        def _pref():
            for cp in copies(s + 1):
                cp.start()

        for cp in copies(s):
            cp.wait()
        slot = lax.rem(s, 2)

        ts = ts_b[slot]
        osc = os_b[slot]
        sm = jnp.sum(osc, axis=1, keepdims=True)
        p = jnp.clip(osc / sm, _EPS, 1.0 - _EPS)
        ce_c = jnp.sum(ts * jnp.log(p))

        col = lax.broadcasted_iota(jnp.int32, ts.shape, 1)
        tsf = ts * (col >= 1).astype(jnp.float32)
        pos_c = jnp.sum(tsf)

        x = td_b[slot] - od_b[slot]
        ax = jnp.abs(x)
        mn = jnp.minimum(ax, 1.0)
        h = mn * (ax - 0.5 * mn)

        c4 = ts.shape[1]
        rj = lax.broadcasted_iota(jnp.int32, (4 * c4, c4), 0) // 4
        cj = lax.broadcasted_iota(jnp.int32, (4 * c4, c4), 1)
        m = (rj == cj).astype(jnp.float32)
        h4 = lax.dot(h, m, preferred_element_type=jnp.float32)
        reg_c = jnp.sum(tsf * h4)

        acc_ref[0] += ce_c
        acc_ref[1] += pos_c
        acc_ref[2] += reg_c
        return carry

    lax.fori_loop(0, total, body, 0)
    out_ref[0] = acc_ref[0]
    out_ref[1] = acc_ref[1]
    out_ref[2] = acc_ref[2]


@jax.jit
def kernel(target_deltas, target_scores, output_deltas, output_scores):
    b, n, c = target_scores.shape
    rows = b * n

    acc = pl.pallas_call(
        _loss_kernel,
        in_specs=[pl.BlockSpec(memory_space=pl.MemorySpace.ANY)] * 4,
        out_specs=pl.BlockSpec(memory_space=pltpu.SMEM),
        out_shape=jax.ShapeDtypeStruct((3,), jnp.float32),
        scratch_shapes=[
            pltpu.VMEM((2, _BLK, 4 * c), jnp.float32),
            pltpu.VMEM((2, _BLK, c), jnp.float32),
            pltpu.VMEM((2, _BLK, 4 * c), jnp.float32),
            pltpu.VMEM((2, _BLK, c), jnp.float32),
            pltpu.SemaphoreType.DMA((2, 4)),
            pltpu.SMEM((3,), jnp.float32),
        ],
    )(target_deltas, target_scores, output_deltas, output_scores)

    cls_loss = -acc[0] / rows
    reg_loss = acc[2] / jnp.maximum(_EPS, acc[1])
    return cls_loss + reg_loss


# speedup vs baseline: 1.0288x; 1.0288x over previous
import jax
import jax.numpy as jnp
from jax import lax
from jax.experimental import pallas as pl
from jax.experimental.pallas import tpu as pltpu

_EPS = 1e-7


def _loss_kernel(td_ref, ts_ref, od_ref, os_ref, out_ref, acc_ref):
    i = pl.program_id(0) * pl.num_programs(1) + pl.program_id(1)
    g = pl.num_programs(0) * pl.num_programs(1)

    ts = ts_ref[0]
    osc = os_ref[0]
    s = jnp.sum(osc, axis=1, keepdims=True)
    p = jnp.clip(osc / s, _EPS, 1.0 - _EPS)
    ce_c = jnp.sum(ts * jnp.log(p))

    col = lax.broadcasted_iota(jnp.int32, ts.shape, 1)
    tsf = ts * (col >= 1).astype(jnp.float32)
    pos_c = jnp.sum(tsf)

    x = td_ref[0] - od_ref[0]
    ax = jnp.abs(x)
    mn = jnp.minimum(ax, 1.0)
    h = mn * (ax - 0.5 * mn)

    c4 = ts.shape[1]
    rj = lax.broadcasted_iota(jnp.int32, (4 * c4, c4), 0) // 4
    cj = lax.broadcasted_iota(jnp.int32, (4 * c4, c4), 1)
    m = (rj == cj).astype(jnp.float32)
    h4 = lax.dot(h, m, preferred_element_type=jnp.float32)
    reg_c = jnp.sum(tsf * h4)

    @pl.when(i == 0)
    def _init():
        acc_ref[0] = 0.0
        acc_ref[1] = 0.0
        acc_ref[2] = 0.0

    acc_ref[0] += ce_c
    acc_ref[1] += pos_c
    acc_ref[2] += reg_c

    @pl.when(i == g - 1)
    def _fin():
        out_ref[...] = jnp.concatenate(
            [jnp.reshape(acc_ref[0], (1, 1)),
             jnp.reshape(acc_ref[1], (1, 1)),
             jnp.reshape(acc_ref[2], (1, 1))], axis=1)


@jax.jit
def kernel(target_deltas, target_scores, output_deltas, output_scores):
    b, n, c = target_scores.shape
    rows = b * n

    blk = 3000
    grid = (b, n // blk)
    acc = pl.pallas_call(
        _loss_kernel,
        grid=grid,
        in_specs=[
            pl.BlockSpec((1, blk, 4 * c), lambda i, j: (i, j, 0)),
            pl.BlockSpec((1, blk, c), lambda i, j: (i, j, 0)),
            pl.BlockSpec((1, blk, 4 * c), lambda i, j: (i, j, 0)),
            pl.BlockSpec((1, blk, c), lambda i, j: (i, j, 0)),
        ],
        out_specs=pl.BlockSpec((1, 3), lambda i, j: (0, 0)),
        out_shape=jax.ShapeDtypeStruct((1, 3), jnp.float32),
        scratch_shapes=[pltpu.SMEM((3,), jnp.float32)],
    )(target_deltas, target_scores, output_deltas, output_scores)

    cls_loss = -acc[0, 0] / rows
    reg_loss = acc[0, 2] / jnp.maximum(_EPS, acc[0, 1])
    return cls_loss + reg_loss
